# Initial kernel scaffold; baseline (speedup 1.0000x reference)
import jax
import jax.numpy as jnp
from jax.experimental import pallas as pl


def kernel(x, router_w, gating_w, linear_w, per_expert_scale, router_scale):
    raise NotImplementedError("write your pallas kernel here")


# speedup vs baseline: 3.2159x; 3.2159x over previous
import functools

import jax
import jax.numpy as jnp
from jax import lax
from jax.experimental import pallas as pl
from jax.experimental.pallas import tpu as pltpu

_BM = 512
_BH = 512


def _ffn_body(meta_ref, xs_ref, wg1_ref, wg2_ref, wl_ref, ws_ref, out_ref):
    w = pl.program_id(0)
    h = pl.program_id(1)
    tile = meta_ref[0, w]
    start = meta_ref[2, w]
    end = meta_ref[3, w]
    first = meta_ref[4, w]

    rows = tile * _BM + lax.broadcasted_iota(jnp.int32, (_BM, 1), 0)
    mask = (rows >= start) & (rows < end)
    xm = jnp.where(mask, xs_ref[...], 0.0)

    x1 = lax.dot_general(xm, wg1_ref[0], (((1,), (1,)), ((), ())),
                         preferred_element_type=jnp.float32)
    x2 = lax.dot_general(xm, wg2_ref[0], (((1,), (1,)), ((), ())),
                         preferred_element_type=jnp.float32)
    act = jax.nn.gelu(x1) * x2
    contrib = lax.dot_general(act, wl_ref[0], (((1,), (0,)), ((), ())),
                              preferred_element_type=jnp.float32)
    contrib = contrib * ws_ref[0, 0, :][:, None]

    init = (h == 0) & (first == 1)

    @pl.when(init)
    def _():
        out_ref[...] = contrib

    @pl.when(jnp.logical_not(init))
    def _():
        out_ref[...] = out_ref[...] + contrib


def _grouped_ffn(sorted_xs, wg1, wg2, wl, ws, meta, n_rows, feats, hidden):
    n_tiles = n_rows // _BM
    nh = hidden // _BH
    n_items = meta.shape[1]
    ws3 = ws.reshape(n_tiles, 1, _BM)
    grid_spec = pltpu.PrefetchScalarGridSpec(
        num_scalar_prefetch=1,
        grid=(n_items, nh),
        in_specs=[
            pl.BlockSpec((_BM, feats), lambda w, h, m: (m[0, w], 0)),
            pl.BlockSpec((1, _BH, feats), lambda w, h, m: (m[1, w], h, 0)),
            pl.BlockSpec((1, _BH, feats), lambda w, h, m: (m[1, w], h, 0)),
            pl.BlockSpec((1, _BH, feats), lambda w, h, m: (m[1, w], h, 0)),
            pl.BlockSpec((1, 1, _BM), lambda w, h, m: (m[0, w], 0, 0)),
        ],
        out_specs=pl.BlockSpec((_BM, feats), lambda w, h, m: (m[0, w], 0)),
    )
    return pl.pallas_call(
        _ffn_body,
        grid_spec=grid_spec,
        out_shape=jax.ShapeDtypeStruct((n_rows, feats), jnp.float32),
        compiler_params=pltpu.CompilerParams(
            dimension_semantics=("arbitrary", "arbitrary"),
        ),
    )(meta, sorted_xs, wg1, wg2, wl, ws3)


def _work_items(counts, n_rows, n_experts):
    n_tiles = n_rows // _BM
    n_items = n_tiles + n_experts - 1
    ends = jnp.cumsum(counts)
    starts = ends - counts
    first_tile = starts // _BM
    last_tile = jnp.maximum(ends - 1, 0) // _BM
    ntiles = jnp.where(counts > 0, last_tile - first_tile + 1, 0)
    cumw = jnp.cumsum(ntiles)
    total = cumw[-1]
    item_e = jnp.repeat(jnp.arange(n_experts), ntiles,
                        total_repeat_length=n_items)
    idx = jnp.arange(n_items)
    valid = idx < total
    off = idx - (cumw - ntiles)[item_e]
    tile_item = jnp.where(valid, first_tile[item_e] + off, n_tiles - 1)
    start_item = jnp.where(valid, starts[item_e], 0)
    end_item = jnp.where(valid, ends[item_e], 0)
    prev_tile = jnp.concatenate([jnp.full((1,), -1, tile_item.dtype),
                                 tile_item[:-1]])
    first_item = (tile_item != prev_tile).astype(jnp.int32)
    return jnp.stack([tile_item, item_e, start_item, end_item,
                      first_item]).astype(jnp.int32)


def kernel(x, router_w, gating_w, linear_w, per_expert_scale, router_scale):
    g, s, feats = x.shape
    n_experts = router_w.shape[1]
    hidden = linear_w.shape[1]
    k = 2
    x2d = x.reshape(-1, feats)
    n_tok = x2d.shape[0]
    n_rows = n_tok * k

    var = jnp.mean(jnp.square(x2d), axis=-1, keepdims=True)
    ri = x2d * lax.rsqrt(var + 1e-6)
    ri = ri * lax.rsqrt(jnp.float32(feats)) * router_scale
    logits = ri @ router_w
    top_v, choices = lax.top_k(logits, k)
    cw = jax.nn.softmax(top_v, axis=-1)

    cflat = choices.reshape(-1)
    order = jnp.argsort(cflat, stable=True)
    inv = jnp.argsort(order)
    counts = jnp.sum(jax.nn.one_hot(cflat, n_experts, dtype=jnp.int32), axis=0)
    sorted_xs = x2d[order // k]
    ws = cw.reshape(-1)[order]

    meta = _work_items(counts, n_rows, n_experts)

    wg1 = gating_w[:, 0]
    wg2 = gating_w[:, 1]
    wl = linear_w * per_expert_scale[:, None, None]
    y = _grouped_ffn(sorted_xs, wg1, wg2, wl, ws, meta, n_rows, feats, hidden)

    slots = inv.reshape(n_tok, k)
    out2d = y[slots[:, 0]] + y[slots[:, 1]]
    return out2d.reshape(g, s, feats)


# speedup vs baseline: 3.7982x; 1.1811x over previous
import functools

import jax
import jax.numpy as jnp
from jax import lax
from jax.experimental import pallas as pl
from jax.experimental.pallas import tpu as pltpu

_BM = 512
_BH = 512


def _ffn_body(meta_ref, xs_ref, wg1_ref, wg2_ref, wl_ref, ws_ref, out_ref):
    w = pl.program_id(0)
    h = pl.program_id(1)
    tile = meta_ref[0, w]
    start = meta_ref[2, w]
    end = meta_ref[3, w]
    first = meta_ref[4, w]

    rows = tile * _BM + lax.broadcasted_iota(jnp.int32, (_BM, 1), 0)
    mask = (rows >= start) & (rows < end)
    xm = jnp.where(mask, xs_ref[...], jnp.bfloat16(0.0))

    x1 = lax.dot_general(xm, wg1_ref[0], (((1,), (1,)), ((), ())),
                         preferred_element_type=jnp.float32)
    x2 = lax.dot_general(xm, wg2_ref[0], (((1,), (1,)), ((), ())),
                         preferred_element_type=jnp.float32)
    act = (jax.nn.gelu(x1) * x2).astype(jnp.bfloat16)
    contrib = lax.dot_general(act, wl_ref[0], (((1,), (0,)), ((), ())),
                              preferred_element_type=jnp.float32)
    contrib = contrib * ws_ref[0, 0, :][:, None]

    init = (h == 0) & (first == 1)

    @pl.when(init)
    def _():
        out_ref[...] = contrib

    @pl.when(jnp.logical_not(init))
    def _():
        out_ref[...] = out_ref[...] + contrib


def _grouped_ffn(sorted_xs, wg1, wg2, wl, ws, meta, n_rows, feats, hidden):
    n_tiles = n_rows // _BM
    nh = hidden // _BH
    n_items = meta.shape[1]
    ws3 = ws.reshape(n_tiles, 1, _BM)
    grid_spec = pltpu.PrefetchScalarGridSpec(
        num_scalar_prefetch=1,
        grid=(n_items, nh),
        in_specs=[
            pl.BlockSpec((_BM, feats), lambda w, h, m: (m[0, w], 0)),
            pl.BlockSpec((1, _BH, feats), lambda w, h, m: (m[1, w], h, 0)),
            pl.BlockSpec((1, _BH, feats), lambda w, h, m: (m[1, w], h, 0)),
            pl.BlockSpec((1, _BH, feats), lambda w, h, m: (m[1, w], h, 0)),
            pl.BlockSpec((1, 1, _BM), lambda w, h, m: (m[0, w], 0, 0)),
        ],
        out_specs=pl.BlockSpec((_BM, feats), lambda w, h, m: (m[0, w], 0)),
    )
    return pl.pallas_call(
        _ffn_body,
        grid_spec=grid_spec,
        out_shape=jax.ShapeDtypeStruct((n_rows, feats), jnp.float32),
        compiler_params=pltpu.CompilerParams(
            dimension_semantics=("arbitrary", "arbitrary"),
        ),
    )(meta, sorted_xs, wg1, wg2, wl, ws3)


def _work_items(counts, n_rows, n_experts):
    n_tiles = n_rows // _BM
    n_items = n_tiles + n_experts - 1
    ends = jnp.cumsum(counts)
    starts = ends - counts
    first_tile = starts // _BM
    last_tile = jnp.maximum(ends - 1, 0) // _BM
    ntiles = jnp.where(counts > 0, last_tile - first_tile + 1, 0)
    cumw = jnp.cumsum(ntiles)
    total = cumw[-1]
    item_e = jnp.repeat(jnp.arange(n_experts), ntiles,
                        total_repeat_length=n_items)
    idx = jnp.arange(n_items)
    valid = idx < total
    off = idx - (cumw - ntiles)[item_e]
    tile_item = jnp.where(valid, first_tile[item_e] + off, n_tiles - 1)
    start_item = jnp.where(valid, starts[item_e], 0)
    end_item = jnp.where(valid, ends[item_e], 0)
    prev_tile = jnp.concatenate([jnp.full((1,), -1, tile_item.dtype),
                                 tile_item[:-1]])
    first_item = (tile_item != prev_tile).astype(jnp.int32)
    return jnp.stack([tile_item, item_e, start_item, end_item,
                      first_item]).astype(jnp.int32)


def kernel(x, router_w, gating_w, linear_w, per_expert_scale, router_scale):
    g, s, feats = x.shape
    n_experts = router_w.shape[1]
    hidden = linear_w.shape[1]
    k = 2
    x2d = x.reshape(-1, feats)
    n_tok = x2d.shape[0]
    n_rows = n_tok * k

    var = jnp.mean(jnp.square(x2d), axis=-1, keepdims=True)
    ri = x2d * lax.rsqrt(var + 1e-6)
    ri = ri * lax.rsqrt(jnp.float32(feats)) * router_scale
    logits = ri @ router_w
    top_v, choices = lax.top_k(logits, k)
    cw = jax.nn.softmax(top_v, axis=-1)

    cflat = choices.reshape(-1)
    order = jnp.argsort(cflat, stable=True)
    inv = jnp.argsort(order)
    counts = jnp.sum(jax.nn.one_hot(cflat, n_experts, dtype=jnp.int32), axis=0)
    sorted_xs = x2d.astype(jnp.bfloat16)[order // k]
    ws = cw.reshape(-1)[order]

    meta = _work_items(counts, n_rows, n_experts)

    wg1 = gating_w[:, 0].astype(jnp.bfloat16)
    wg2 = gating_w[:, 1].astype(jnp.bfloat16)
    wl = (linear_w * per_expert_scale[:, None, None]).astype(jnp.bfloat16)
    y = _grouped_ffn(sorted_xs, wg1, wg2, wl, ws, meta, n_rows, feats, hidden)

    slots = inv.reshape(n_tok, k)
    out2d = y[slots[:, 0]] + y[slots[:, 1]]
    return out2d.reshape(g, s, feats)
